# Initial kernel scaffold; baseline (speedup 1.0000x reference)
import functools

import jax
import jax.numpy as jnp
from jax import lax
from jax.experimental import pallas as pl
from jax.experimental.pallas import tpu as pltpu
from jax.experimental.pallas import tpu_sc as plsc

N = 10000
E = 320000
D = 128

NC = 2
NS = 16
NW = NC * NS

K = 80
NCH = E // (NW * K)
ROWS_PER_TILE = N // NS


def _make_agg(d):
    mesh = plsc.VectorSubcoreMesh(core_axis_name="c", subcore_axis_name="s")

    @functools.partial(
        pl.kernel,
        mesh=mesh,
        out_type=jax.ShapeDtypeStruct((NC, N, d), jnp.float32),
        scratch_types=[
            pltpu.VMEM((NCH, K), jnp.int32),
            pltpu.VMEM((NCH, K), jnp.int32),
            pltpu.VMEM((K, d), jnp.float32),
            pltpu.VMEM_SHARED((N, d), jnp.float32),
            pltpu.SemaphoreType.DMA,
        ],
    )
    def agg(h_hbm, src_hbm, dst_hbm, out_hbm, src_v, dst_v, rows_v, acc_sh, sem):
        c = lax.axis_index("c")
        s = lax.axis_index("s")
        wid = c * NS + s
        rbase = s * ROWS_PER_TILE

        @pl.when(c == 0)
        def _():
            pltpu.sync_copy(
                h_hbm.at[pl.ds(rbase, ROWS_PER_TILE)],
                acc_sh.at[pl.ds(rbase, ROWS_PER_TILE)],
            )

        @pl.when(c != 0)
        def _():
            zrows = 25
            for i in range(zrows):
                for j in range(d // 16):
                    rows_v[i, pl.ds(j * 16, 16)] = jnp.zeros((16,), jnp.float32)
            for t in range(ROWS_PER_TILE // zrows):
                pltpu.sync_copy(
                    rows_v.at[pl.ds(0, zrows)],
                    acc_sh.at[pl.ds(rbase + t * zrows, zrows)],
                )

        pltpu.sync_copy(src_hbm.at[wid], src_v)
        pltpu.sync_copy(dst_hbm.at[wid], dst_v)
        plsc.subcore_barrier()

        def body(j, carry):
            pltpu.async_copy(h_hbm.at[src_v.at[j]], rows_v, sem).wait()
            pltpu.sync_copy(rows_v, acc_sh.at[dst_v.at[j]], add=True)
            return carry

        lax.fori_loop(0, NCH, body, 0, unroll=False)
        plsc.subcore_barrier()

        pltpu.sync_copy(
            acc_sh.at[pl.ds(rbase, ROWS_PER_TILE)],
            out_hbm.at[c, pl.ds(rbase, ROWS_PER_TILE)],
        )

    return agg


_agg128 = _make_agg(128)
_agg64 = _make_agg(64)


_BR = 500


def _lin0_body(x_ref, w_ref, b_ref, o_ref):
    o_ref[...] = jnp.dot(x_ref[...], w_ref[...],
                         preferred_element_type=jnp.float32) + b_ref[...]


def _lin0(x, w, b):
    return pl.pallas_call(
        _lin0_body,
        grid=(N // _BR,),
        in_specs=[
            pl.BlockSpec((_BR, 128), lambda i: (i, 0)),
            pl.BlockSpec((128, w.shape[1]), lambda i: (0, 0)),
            pl.BlockSpec((1, w.shape[1]), lambda i: (0, 0)),
        ],
        out_specs=pl.BlockSpec((_BR, w.shape[1]), lambda i: (i, 0)),
        out_shape=jax.ShapeDtypeStruct((N, w.shape[1]), jnp.float32),
    )(x, w, b.reshape(1, -1))


def _combine_lin_body(p_ref, w_ref, b_ref, o_ref):
    x = jnp.maximum(p_ref[0] + p_ref[1], 0.0)
    o_ref[...] = jnp.dot(x, w_ref[...],
                         preferred_element_type=jnp.float32) + b_ref[...]


def _combine_lin(p, w, b):
    dout = w.shape[1]
    return pl.pallas_call(
        _combine_lin_body,
        grid=(N // _BR,),
        in_specs=[
            pl.BlockSpec((2, _BR, 128), lambda i: (0, i, 0)),
            pl.BlockSpec((128, dout), lambda i: (0, 0)),
            pl.BlockSpec((1, dout), lambda i: (0, 0)),
        ],
        out_specs=pl.BlockSpec((_BR, dout), lambda i: (i, 0)),
        out_shape=jax.ShapeDtypeStruct((N, dout), jnp.float32),
    )(p, w, b.reshape(1, -1))


def _loss_body(p_ref, lab_ref, o_ref, acc_ref):
    i = pl.program_id(0)

    @pl.when(i == 0)
    def _():
        acc_ref[0] = jnp.zeros((1,), jnp.float32)

    z = p_ref[0] + p_ref[1]
    cols = lax.broadcasted_iota(jnp.int32, z.shape, 1)
    valid = cols < 40
    neg = jnp.full_like(z, -jnp.inf)
    zm = jnp.where(valid, z, neg)
    m = jnp.max(zm, axis=1, keepdims=True)
    se = jnp.sum(jnp.where(valid, jnp.exp(z - m), 0.0), axis=1, keepdims=True)
    lse = m + jnp.log(se)
    lab = lab_ref[...]
    zlab = jnp.sum(jnp.where(cols == lab, z, 0.0), axis=1, keepdims=True)
    acc_ref[0] += jnp.sum(lse - zlab)

    @pl.when(i == pl.num_programs(0) - 1)
    def _():
        o_ref[0] = acc_ref[0]


def _loss(p, labels):
    return pl.pallas_call(
        _loss_body,
        grid=(N // _BR,),
        in_specs=[
            pl.BlockSpec((2, _BR, 64), lambda i: (0, i, 0)),
            pl.BlockSpec((_BR, 1), lambda i: (i, 0)),
        ],
        out_specs=pl.BlockSpec(memory_space=pltpu.SMEM),
        out_shape=jax.ShapeDtypeStruct((1,), jnp.float32),
        scratch_shapes=[pltpu.SMEM((1,), jnp.float32)],
    )(p, labels.reshape(N, 1))[0]


@jax.jit
def kernel(features, labels, edge_index, W0, b0, W1, b1, W2, b2):
    src = edge_index[0].astype(jnp.int32).reshape(NW, NCH, K)
    dst = edge_index[1].astype(jnp.int32).reshape(NW, NCH, K)

    h = _lin0(features, W0, b0)
    p = _agg128(h, src, dst)
    h = _combine_lin(p, W1, b1)
    p = _agg128(h, src, dst)
    W2p = jnp.pad(W2, ((0, 0), (0, 24)))
    b2p = jnp.pad(b2, (0, 24))
    h = _combine_lin(p, W2p, b2p)
    p = _agg64(h, src, dst)
    return _loss(p, labels.astype(jnp.int32))


# speedup vs baseline: 6.7673x; 6.7673x over previous
import functools

import jax
import jax.numpy as jnp
from jax import lax
from jax.experimental import pallas as pl
from jax.experimental.pallas import tpu as pltpu
from jax.experimental.pallas import tpu_sc as plsc

N = 10000
E = 320000
D = 128

NC = 2
NS = 16
NW = NC * NS

K = 80
NCH = E // (NW * K)
ROWS_PER_TILE = N // NS


def _make_agg(d):
    mesh = plsc.VectorSubcoreMesh(core_axis_name="c", subcore_axis_name="s")

    @functools.partial(
        pl.kernel,
        mesh=mesh,
        out_type=jax.ShapeDtypeStruct((NC, NS, ROWS_PER_TILE, d), jnp.float32),
        scratch_types=[
            pltpu.VMEM((NCH, K), jnp.int32),
            pltpu.VMEM((NCH, K), jnp.int32),
            pltpu.VMEM((K, d), jnp.float32),
            pltpu.VMEM_SHARED((N, d), jnp.float32),
            pltpu.SemaphoreType.DMA,
        ],
    )
    def agg(h_hbm, h3_hbm, src_hbm, dst_hbm, out_hbm,
            src_v, dst_v, rows_v, acc_sh, sem):
        c = lax.axis_index("c")
        s = lax.axis_index("s")
        wid = c * NS + s
        rbase = s * ROWS_PER_TILE

        @pl.when(c == 0)
        def _():
            pltpu.sync_copy(
                h3_hbm.at[s],
                acc_sh.at[pl.ds(rbase, ROWS_PER_TILE)],
            )

        @pl.when(c != 0)
        def _():
            zrows = 25
            for i in range(zrows):
                for j in range(d // 16):
                    rows_v[i, pl.ds(j * 16, 16)] = jnp.zeros((16,), jnp.float32)
            for t in range(ROWS_PER_TILE // zrows):
                pltpu.sync_copy(
                    rows_v.at[pl.ds(0, zrows)],
                    acc_sh.at[pl.ds(rbase + t * zrows, zrows)],
                )

        pltpu.sync_copy(src_hbm.at[wid], src_v)
        pltpu.sync_copy(dst_hbm.at[wid], dst_v)
        plsc.subcore_barrier()

        def body(j, carry):
            pltpu.async_copy(h_hbm.at[src_v.at[j]], rows_v, sem).wait()
            pltpu.sync_copy(rows_v, acc_sh.at[dst_v.at[j]], add=True)
            return carry

        lax.fori_loop(0, NCH, body, 0, unroll=False)
        plsc.subcore_barrier()

        pltpu.sync_copy(
            acc_sh.at[pl.ds(rbase, ROWS_PER_TILE)],
            out_hbm.at[c, s],
        )

    return agg


_agg128 = _make_agg(128)


_BR = 1000


def _lin0_body(x_ref, w_ref, b_ref, o_ref):
    o_ref[...] = jnp.dot(x_ref[...], w_ref[...],
                         preferred_element_type=jnp.float32) + b_ref[...]


def _lin0(x, w, b):
    return pl.pallas_call(
        _lin0_body,
        grid=(N // _BR,),
        in_specs=[
            pl.BlockSpec((_BR, 128), lambda i: (i, 0)),
            pl.BlockSpec((128, w.shape[1]), lambda i: (0, 0)),
            pl.BlockSpec((1, w.shape[1]), lambda i: (0, 0)),
        ],
        out_specs=pl.BlockSpec((_BR, w.shape[1]), lambda i: (i, 0)),
        out_shape=jax.ShapeDtypeStruct((N, w.shape[1]), jnp.float32),
    )(x, w, b.reshape(1, -1))


def _combine_lin_body(p_ref, w_ref, b_ref, o_ref):
    x = jnp.maximum(p_ref[0] + p_ref[1], 0.0)
    o_ref[...] = jnp.dot(x, w_ref[...],
                         preferred_element_type=jnp.float32) + b_ref[...]


def _combine_lin(p, w, b):
    dout = w.shape[1]
    return pl.pallas_call(
        _combine_lin_body,
        grid=(N // _BR,),
        in_specs=[
            pl.BlockSpec((2, _BR, 128), lambda i: (0, i, 0)),
            pl.BlockSpec((128, dout), lambda i: (0, 0)),
            pl.BlockSpec((1, dout), lambda i: (0, 0)),
        ],
        out_specs=pl.BlockSpec((_BR, dout), lambda i: (i, 0)),
        out_shape=jax.ShapeDtypeStruct((N, dout), jnp.float32),
    )(p, w, b.reshape(1, -1))


def _loss_body(p_ref, lab_ref, o_ref, acc_ref):
    i = pl.program_id(0)

    @pl.when(i == 0)
    def _():
        acc_ref[0] = 0.0

    z = p_ref[0] + p_ref[1]
    cols = lax.broadcasted_iota(jnp.int32, z.shape, 1)
    valid = cols < 40
    neg = jnp.full_like(z, -jnp.inf)
    zm = jnp.where(valid, z, neg)
    m = jnp.max(zm, axis=1, keepdims=True)
    se = jnp.sum(jnp.where(valid, jnp.exp(z - m), 0.0), axis=1, keepdims=True)
    lse = m + jnp.log(se)
    lab = lab_ref[...]
    zlab = jnp.sum(jnp.where(cols == lab, z, 0.0), axis=1, keepdims=True)
    acc_ref[0] += jnp.sum(lse - zlab)

    @pl.when(i == pl.num_programs(0) - 1)
    def _():
        o_ref[0] = acc_ref[0]


def _loss(p, labels):
    return pl.pallas_call(
        _loss_body,
        grid=(N // _BR,),
        in_specs=[
            pl.BlockSpec((2, _BR, 128), lambda i: (0, i, 0)),
            pl.BlockSpec((_BR, 1), lambda i: (i, 0)),
        ],
        out_specs=pl.BlockSpec(memory_space=pltpu.SMEM),
        out_shape=jax.ShapeDtypeStruct((1,), jnp.float32),
        scratch_shapes=[pltpu.SMEM((1,), jnp.float32)],
    )(p, labels.reshape(N, 1))[0]


@jax.jit
def kernel(features, labels, edge_index, W0, b0, W1, b1, W2, b2):
    src = edge_index[0].astype(jnp.int32).reshape(NW, NCH, K)
    dst = edge_index[1].astype(jnp.int32).reshape(NW, NCH, K)

    def agg(fn, h, d):
        h3 = h.reshape(NS, ROWS_PER_TILE, d)
        return fn(h, h3, src, dst).reshape(NC, N, d)

    h = _lin0(features, W0, b0)
    p = agg(_agg128, h, 128)
    h = _combine_lin(p, W1, b1)
    p = agg(_agg128, h, 128)
    W2p = jnp.pad(W2, ((0, 0), (0, 88)))
    b2p = jnp.pad(b2, (0, 88))
    h = _combine_lin(p, W2p, b2p)
    p = agg(_agg128, h, 128)
    return _loss(p, labels.astype(jnp.int32))
